# baseline (speedup 1.0000x reference)
import functools

import jax
import jax.numpy as jnp
from jax import lax
from jax.experimental import pallas as pl
from jax.experimental.pallas import tpu as pltpu
from jax.experimental.pallas import tpu_sc as plsc

_LANES = 16


def _pooled_sc_kernel(indices, emb_table):
    B, L = indices.shape
    V, D = emb_table.shape

    info = plsc.get_sparse_core_info()
    NC, NS = info.num_cores, info.num_subcores
    NW = NC * NS

    rows_per_w = B // NW
    C = 64
    n_groups = rows_per_w // C
    n_sub = C // _LANES

    mesh = plsc.VectorSubcoreMesh(core_axis_name="c", subcore_axis_name="s")

    def body(idx_hbm, table_hbm, out_hbm, table_v, idx_v, out_v,
             isem0, isem1, osem0, osem1):
        wid = lax.axis_index("s") * NC + lax.axis_index("c")
        base_row = wid * rows_per_w
        isems = (isem0, isem1)
        osems = (osem0, osem1)

        idx_cp = [None] * n_groups
        idx_cp[0] = pltpu.async_copy(
            idx_hbm.at[pl.ds(base_row, C), :], idx_v.at[0], isems[0]
        )
        pltpu.sync_copy(table_hbm, table_v)

        row_iota = lax.iota(jnp.int32, _LANES)
        zeros = tuple(jnp.zeros((_LANES,), jnp.float32) for _ in range(D))
        inv_l = jnp.float32(1.0 / L)

        out_cp = [None] * n_groups
        for g in range(n_groups):
            buf = g % 2
            row0 = base_row + g * C
            if g + 1 < n_groups:
                idx_cp[g + 1] = pltpu.async_copy(
                    idx_hbm.at[pl.ds(row0 + C, C), :],
                    idx_v.at[1 - buf],
                    isems[1 - buf],
                )
            idx_cp[g].wait()
            if g >= 2:
                out_cp[g - 2].wait()

            def sub_body(s, carry2):
                rvec = s * _LANES + row_iota

                def l_body(l, carry):
                    lvec, accs = carry
                    vidx = plsc.load_gather(idx_v.at[buf], [rvec, lvec])
                    tbase = vidx * D
                    return (
                        lvec + 1,
                        tuple(
                            accs[d] + plsc.load_gather(table_v, [tbase + d])
                            for d in range(D)
                        ),
                    )

                _, accs = lax.fori_loop(
                    0, L, l_body, (jnp.zeros((_LANES,), jnp.int32), zeros),
                    unroll=4,
                )
                for d in range(D):
                    dvec = jnp.full((_LANES,), d, jnp.int32)
                    plsc.store_scatter(out_v.at[buf], [rvec, dvec], accs[d] * inv_l)
                return carry2

            lax.fori_loop(0, n_sub, sub_body, 0)
            out_cp[g] = pltpu.async_copy(
                out_v.at[buf], out_hbm.at[pl.ds(row0, C), :], osems[buf]
            )
        for g in range(max(0, n_groups - 2), n_groups):
            out_cp[g].wait()

    return pl.kernel(
        body,
        out_type=jax.ShapeDtypeStruct((B, D), jnp.float32),
        mesh=mesh,
        compiler_params=pltpu.CompilerParams(needs_layout_passes=False),
        scratch_types=[
            pltpu.VMEM((V * D,), jnp.float32),
            pltpu.VMEM((2, C, L), jnp.int32),
            pltpu.VMEM((2, C, D), jnp.float32),
            pltpu.SemaphoreType.DMA,
            pltpu.SemaphoreType.DMA,
            pltpu.SemaphoreType.DMA,
            pltpu.SemaphoreType.DMA,
        ],
    )(indices, emb_table.reshape(V * D))


def _mlp_body(x_ref, w1_ref, b1_ref, w2_ref, b2_ref, w3_ref, b3_ref, o_ref):
    x = x_ref[...]
    h1 = jnp.maximum(
        jnp.dot(x, w1_ref[...], preferred_element_type=jnp.float32) + b1_ref[...], 0.0
    )
    h2 = jnp.maximum(
        jnp.dot(h1, w2_ref[...], preferred_element_type=jnp.float32) + b2_ref[...], 0.0
    )
    o_ref[...] = jax.nn.sigmoid(
        jnp.dot(h2, w3_ref[...], preferred_element_type=jnp.float32) + b3_ref[...]
    )


def _mlp_tc(pooled, W1, b1, W2, b2, W3, b3):
    B, D = pooled.shape
    H1 = W1.shape[1]
    H2 = W2.shape[1]
    TB = 2048
    grid = (B // TB,)
    b1r, b2r, b3r = b1.reshape(1, H1), b2.reshape(1, H2), b3.reshape(1, 1)
    fixed = lambda i: (0, 0)
    return pl.pallas_call(
        _mlp_body,
        grid=grid,
        in_specs=[
            pl.BlockSpec((TB, D), lambda i: (i, 0)),
            pl.BlockSpec((D, H1), fixed),
            pl.BlockSpec((1, H1), fixed),
            pl.BlockSpec((H1, H2), fixed),
            pl.BlockSpec((1, H2), fixed),
            pl.BlockSpec((H2, 1), fixed),
            pl.BlockSpec((1, 1), fixed),
        ],
        out_specs=pl.BlockSpec((TB, 1), lambda i: (i, 0)),
        out_shape=jax.ShapeDtypeStruct((B, 1), jnp.float32),
    )(pooled, W1, b1r, W2, b2r, W3, b3r)


def kernel(indices, emb_table, W1, b1, W2, b2, W3, b3):
    pooled = _pooled_sc_kernel(indices, emb_table)
    return _mlp_tc(pooled, W1, b1, W2, b2, W3, b3)


# speedup vs baseline: 1.0728x; 1.0728x over previous
import functools

import jax
import jax.numpy as jnp
from jax import lax
from jax.experimental import pallas as pl
from jax.experimental.pallas import tpu as pltpu
from jax.experimental.pallas import tpu_sc as plsc

_LANES = 16


def _pooled_sc_kernel(indices, emb_table):
    B, L = indices.shape
    V, D = emb_table.shape

    info = plsc.get_sparse_core_info()
    NC, NS = info.num_cores, info.num_subcores
    NW = NC * NS

    rows_per_w = B // NW
    C = 64
    n_groups = rows_per_w // C
    n_sub = C // _LANES

    mesh = plsc.VectorSubcoreMesh(core_axis_name="c", subcore_axis_name="s")

    def body(idx_hbm, table_hbm, out_hbm, table_v, idx_v, out_v,
             isem0, isem1, osem0, osem1):
        wid = lax.axis_index("s") * NC + lax.axis_index("c")
        base_row = wid * rows_per_w
        isems = (isem0, isem1)
        osems = (osem0, osem1)

        idx_cp = [None] * n_groups
        idx_cp[0] = pltpu.async_copy(
            idx_hbm.at[pl.ds(base_row, C), :], idx_v.at[0], isems[0]
        )
        pltpu.sync_copy(table_hbm, table_v)

        row_iota = lax.iota(jnp.int32, _LANES)
        zeros = tuple(jnp.zeros((_LANES,), jnp.float32) for _ in range(D))
        inv_l = jnp.float32(1.0 / L)

        out_cp = [None] * n_groups
        for g in range(n_groups):
            buf = g % 2
            row0 = base_row + g * C
            if g + 1 < n_groups:
                idx_cp[g + 1] = pltpu.async_copy(
                    idx_hbm.at[pl.ds(row0 + C, C), :],
                    idx_v.at[1 - buf],
                    isems[1 - buf],
                )
            idx_cp[g].wait()
            if g >= 2:
                out_cp[g - 2].wait()

            def sub_body(s, carry2):
                rvec = s * _LANES + row_iota

                def l_body(l, accs):
                    lvec = jnp.full((_LANES,), l, jnp.int32)
                    vidx = plsc.load_gather(idx_v.at[buf], [rvec, lvec])
                    tbase = vidx * D
                    return tuple(
                        accs[d] + plsc.load_gather(table_v, [tbase + d])
                        for d in range(D)
                    )

                accs = lax.fori_loop(0, L, l_body, zeros, unroll=2)
                for d in range(D):
                    dvec = jnp.full((_LANES,), d, jnp.int32)
                    plsc.store_scatter(out_v.at[buf], [rvec, dvec], accs[d] * inv_l)
                return carry2

            lax.fori_loop(0, n_sub, sub_body, 0)
            out_cp[g] = pltpu.async_copy(
                out_v.at[buf], out_hbm.at[pl.ds(row0, C), :], osems[buf]
            )
        for g in range(max(0, n_groups - 2), n_groups):
            out_cp[g].wait()

    return pl.kernel(
        body,
        out_type=jax.ShapeDtypeStruct((B, D), jnp.float32),
        mesh=mesh,
        compiler_params=pltpu.CompilerParams(needs_layout_passes=False),
        scratch_types=[
            pltpu.VMEM((V * D,), jnp.float32),
            pltpu.VMEM((2, C, L), jnp.int32),
            pltpu.VMEM((2, C, D), jnp.float32),
            pltpu.SemaphoreType.DMA,
            pltpu.SemaphoreType.DMA,
            pltpu.SemaphoreType.DMA,
            pltpu.SemaphoreType.DMA,
        ],
    )(indices, emb_table.reshape(V * D))


def _mlp_body(x_ref, w1_ref, b1_ref, w2_ref, b2_ref, w3_ref, b3_ref, o_ref):
    x = x_ref[...]
    h1 = jnp.maximum(
        jnp.dot(x, w1_ref[...], preferred_element_type=jnp.float32) + b1_ref[...], 0.0
    )
    h2 = jnp.maximum(
        jnp.dot(h1, w2_ref[...], preferred_element_type=jnp.float32) + b2_ref[...], 0.0
    )
    o_ref[...] = jax.nn.sigmoid(
        jnp.dot(h2, w3_ref[...], preferred_element_type=jnp.float32) + b3_ref[...]
    )


def _mlp_tc(pooled, W1, b1, W2, b2, W3, b3):
    B, D = pooled.shape
    H1 = W1.shape[1]
    H2 = W2.shape[1]
    TB = 2048
    grid = (B // TB,)
    b1r, b2r, b3r = b1.reshape(1, H1), b2.reshape(1, H2), b3.reshape(1, 1)
    fixed = lambda i: (0, 0)
    return pl.pallas_call(
        _mlp_body,
        grid=grid,
        in_specs=[
            pl.BlockSpec((TB, D), lambda i: (i, 0)),
            pl.BlockSpec((D, H1), fixed),
            pl.BlockSpec((1, H1), fixed),
            pl.BlockSpec((H1, H2), fixed),
            pl.BlockSpec((1, H2), fixed),
            pl.BlockSpec((H2, 1), fixed),
            pl.BlockSpec((1, 1), fixed),
        ],
        out_specs=pl.BlockSpec((TB, 1), lambda i: (i, 0)),
        out_shape=jax.ShapeDtypeStruct((B, 1), jnp.float32),
    )(pooled, W1, b1r, W2, b2r, W3, b3r)


def kernel(indices, emb_table, W1, b1, W2, b2, W3, b3):
    pooled = _pooled_sc_kernel(indices, emb_table)
    return _mlp_tc(pooled, W1, b1, W2, b2, W3, b3)
